# baseline (speedup 1.0000x reference)
import jax
import jax.numpy as jnp
from jax.experimental import pallas as pl
from jax.experimental.pallas import tpu as pltpu

_B, _C, _H, _W = 32, 256, 56, 56
_NH = 8
_D = 32
_WS = 7
_S = _WS * _WS
_NWW = _W // _WS
_CLAMP_MAX = 4.6052
_EPS = 1e-12


def _attn_kernel(q_ref, k_ref, v_ref, sc_ref, o_ref):
    def to_tsd(x):
        x = x.reshape(_NH, _D, _WS, _NWW, _WS)
        x = x.transpose(0, 3, 1, 2, 4)
        return x.reshape(_NH * _NWW, _D, _S)

    q = to_tsd(q_ref[0, :, 0])
    k = to_tsd(k_ref[0, :, 0])
    v = to_tsd(v_ref[0, :, 0])

    def l2n(x):
        n = jnp.sqrt(jnp.sum(x * x, axis=1, keepdims=True))
        return x / jnp.maximum(n, _EPS)

    qn = l2n(q)
    kn = l2n(k)

    dots_t = jax.lax.dot_general(
        kn, qn, (((1,), (1,)), ((0,), (0,))),
        preferred_element_type=jnp.float32)
    dots_t = dots_t * sc_ref[...]

    mx = jnp.max(dots_t, axis=1, keepdims=True)
    e = jnp.exp(dots_t - mx)
    denom = jnp.sum(e, axis=1, keepdims=True)
    attn_t = e / denom

    out_t = jax.lax.dot_general(
        v, attn_t, (((2,), (1,)), ((0,), (0,))),
        preferred_element_type=jnp.float32)

    y = out_t.reshape(_NH, _NWW, _D, _WS, _WS)
    y = y.transpose(0, 2, 3, 1, 4)
    o_ref[0, :, 0] = y.reshape(_C, _WS, _W)


def kernel(q, k, v, logit_scale):
    scale = jnp.exp(jnp.minimum(logit_scale, _CLAMP_MAX))
    sc = jnp.broadcast_to(scale.reshape(_NH, 1, 1, 1),
                          (_NH, _NWW, 1, _S)).reshape(_NH * _NWW, 1, _S)

    nr = _H // _WS
    q5 = q.reshape(_B, _C, nr, _WS, _W)
    k5 = k.reshape(_B, _C, nr, _WS, _W)
    v5 = v.reshape(_B, _C, nr, _WS, _W)

    strip = pl.BlockSpec((1, _C, 1, _WS, _W), lambda b, r: (b, 0, r, 0, 0))
    out = pl.pallas_call(
        _attn_kernel,
        out_shape=jax.ShapeDtypeStruct((_B, _C, nr, _WS, _W), jnp.float32),
        grid=(_B, nr),
        in_specs=[strip, strip, strip,
                  pl.BlockSpec((_NH * _NWW, 1, _S), lambda b, r: (0, 0, 0))],
        out_specs=strip,
        compiler_params=pltpu.CompilerParams(
            dimension_semantics=("parallel", "arbitrary")),
        name="win_cos_attn",
    )(q5, k5, v5, sc)
    return out.reshape(_B, _C, _H, _W)


# speedup vs baseline: 2.4737x; 2.4737x over previous
import numpy as np
import jax
import jax.numpy as jnp
from jax.experimental import pallas as pl
from jax.experimental.pallas import tpu as pltpu

_B, _C, _H, _W = 32, 256, 56, 56
_NH = 8
_D = 32
_WS = 7
_S = _WS * _WS
_NWW = _W // _WS
_ROW = _WS * _W
_WPAD = 64
_ROWP = _NWW * _WPAD
_CLAMP_MAX = 4.6052
_EPS = 1e-12
_NEG = -1e30


def _perm_np():
    p = np.zeros((_ROW, _ROWP), dtype=np.float32)
    for i1 in range(_WS):
        for ww in range(_NWW):
            for i2 in range(_WS):
                p[i1 * _W + ww * _WS + i2, ww * _WPAD + i1 * _WS + i2] = 1.0
    return p


def _mask_np():
    r = np.arange(128)[:, None]
    p = np.arange(128)[None, :]
    ok = ((r // _WPAD) == (p // _WPAD)) & ((r % _WPAD) < _S)
    return np.where(ok, 0.0, _NEG).astype(np.float32)


def _attn_kernel(sc_ref, q_ref, k_ref, v_ref, p_ref, pt_ref, lm_ref, o_ref):
    q2 = q_ref[0, :, 0, 0]
    k2 = k_ref[0, :, 0, 0]
    v2 = v_ref[0, :, 0, 0]

    def l2n(x):
        x3 = x.reshape(_NH, _D, _ROW)
        n = jnp.sqrt(jnp.sum(x3 * x3, axis=1, keepdims=True))
        return (x3 / jnp.maximum(n, _EPS)).reshape(_C, _ROW)

    pm = p_ref[...]
    qw = jnp.dot(l2n(q2), pm, preferred_element_type=jnp.float32)
    kw = jnp.dot(l2n(k2), pm, preferred_element_type=jnp.float32)
    vw = jnp.dot(v2, pm, preferred_element_type=jnp.float32)

    lm = lm_ref[...]
    head_rows = []
    for h in range(_NH):
        sc = sc_ref[h]
        qh = qw[h * _D:(h + 1) * _D, :] * sc
        kh = kw[h * _D:(h + 1) * _D, :]
        vh = vw[h * _D:(h + 1) * _D, :]
        lmh = lm - sc
        chunks = []
        for c in range(4):
            sl = slice(c * 128, (c + 1) * 128)
            st = jax.lax.dot_general(
                kh[:, sl], qh[:, sl], (((0,), (0,)), ((), ())),
                preferred_element_type=jnp.float32)
            e = jnp.exp(st + lmh)
            den = jnp.sum(e, axis=0, keepdims=True)
            o_c = jax.lax.dot_general(
                vh[:, sl], e, (((1,), (0,)), ((), ())),
                preferred_element_type=jnp.float32)
            chunks.append(o_c / den)
        head_rows.append(jnp.concatenate(chunks, axis=1))
    outw = jnp.concatenate(head_rows, axis=0)
    o_ref[0, :, 0, 0] = jnp.dot(outw, pt_ref[...],
                                preferred_element_type=jnp.float32)


def kernel(q, k, v, logit_scale):
    sc = jnp.exp(jnp.minimum(logit_scale, _CLAMP_MAX)).reshape(_NH)
    pm = jnp.asarray(_perm_np())
    pt = pm.T
    lm = jnp.asarray(_mask_np())

    nr = _H // _WS
    q5 = q.reshape(_B, _C, nr, 1, _ROW)
    k5 = k.reshape(_B, _C, nr, 1, _ROW)
    v5 = v.reshape(_B, _C, nr, 1, _ROW)

    strip = pl.BlockSpec((1, _C, 1, 1, _ROW), lambda b, r: (b, 0, r, 0, 0))
    fixed = lambda shape: pl.BlockSpec(shape, lambda b, r: tuple([0] * len(shape)))
    out = pl.pallas_call(
        _attn_kernel,
        out_shape=jax.ShapeDtypeStruct((_B, _C, nr, 1, _ROW), jnp.float32),
        grid=(_B, nr),
        in_specs=[pl.BlockSpec(memory_space=pltpu.SMEM),
                  strip, strip, strip,
                  fixed((_ROW, _ROWP)), fixed((_ROWP, _ROW)),
                  fixed((128, 128))],
        out_specs=strip,
        compiler_params=pltpu.CompilerParams(
            dimension_semantics=("parallel", "arbitrary")),
        name="win_cos_attn",
    )(sc, q5, k5, v5, pm, pt, lm)
    return out.reshape(_B, _C, _H, _W)
